# Initial kernel scaffold; baseline (speedup 1.0000x reference)
import jax
import jax.numpy as jnp
from jax.experimental import pallas as pl


def kernel(x, edge_index, batch, W0, b0, g0, be0, m0, v0, W1, b1, g1, be1, m1, v1, W2, b2, g2, be2, m2, v2, cW0, cb0, cW1, cb1):
    raise NotImplementedError("write your pallas kernel here")


# speedup vs baseline: 10.8668x; 10.8668x over previous
import jax
import jax.numpy as jnp
from jax import lax
from jax.experimental import pallas as pl
from jax.experimental.pallas import tpu as pltpu
from jax.experimental.pallas import tpu_sc as plsc

N = 10000
E = 320000
D = 128
H2 = 64
B = 512
EPS = 1e-5

NC = 2
NS = 16
NW = NC * NS

NP = 10240
EPT = E // NW
CH = 80
NCH = EPT // CH
RPT = NP // NS
ZR = 128

BR = 2000
NBLK = N // BR

_MESH = plsc.VectorSubcoreMesh(core_axis_name="c", subcore_axis_name="s",
                               num_cores=NC, num_subcores=NS)


def _sc_deg_body(dst_hbm, degp_hbm, dstv, onesb, zb, degsp):
    c = lax.axis_index("c")
    s = lax.axis_index("s")
    wid = c * NS + s
    one16 = jnp.ones((16,), jnp.float32)
    zero16 = jnp.zeros((16,), jnp.float32)

    @pl.loop(0, CH)
    def _fill(i):
        for j in range(D // 16):
            onesb[i, pl.ds(j * 16, 16)] = one16

    @pl.loop(0, ZR)
    def _zfill(i):
        for j in range(D // 16):
            zb[i, pl.ds(j * 16, 16)] = zero16

    for j in range(RPT // ZR):
        pltpu.sync_copy(zb, degsp.at[pl.ds(s * RPT + j * ZR, ZR)])
    plsc.subcore_barrier()

    base = wid * EPT

    @pl.loop(0, NCH)
    def _chunk(i):
        pltpu.sync_copy(dst_hbm.at[pl.ds(base + i * CH, CH)], dstv)
        pltpu.sync_copy(onesb, degsp.at[dstv], add=True)

    plsc.subcore_barrier()
    for j in range(RPT // ZR):
        sl = pl.ds(s * RPT + j * ZR, ZR)
        pltpu.sync_copy(degsp.at[sl], zb)
        pltpu.sync_copy(zb, degp_hbm.at[c, sl])


_sc_deg = pl.kernel(
    _sc_deg_body,
    out_type=jax.ShapeDtypeStruct((NC, NP, D), jnp.float32),
    mesh=_MESH,
    scratch_types=[
        pltpu.VMEM((CH,), jnp.int32),
        pltpu.VMEM((CH, D), jnp.float32),
        pltpu.VMEM((ZR, D), jnp.float32),
        pltpu.VMEM_SHARED((NP, D), jnp.float32),
    ],
)


def _sc_edge_body(g_hbm, src_hbm, dst_hbm, acc_hbm, srcv, dstv, rows, zb,
                  accsp, sem):
    c = lax.axis_index("c")
    s = lax.axis_index("s")
    wid = c * NS + s
    zero16 = jnp.zeros((16,), jnp.float32)

    @pl.loop(0, ZR)
    def _zfill(i):
        for j in range(D // 16):
            zb[i, pl.ds(j * 16, 16)] = zero16

    for j in range(RPT // ZR):
        pltpu.sync_copy(zb, accsp.at[pl.ds(s * RPT + j * ZR, ZR)])
    plsc.subcore_barrier()

    base = wid * EPT

    @pl.loop(0, NCH)
    def _chunk(i):
        off = base + i * CH
        pltpu.sync_copy(src_hbm.at[pl.ds(off, CH)], srcv)
        pltpu.sync_copy(dst_hbm.at[pl.ds(off, CH)], dstv)
        pltpu.async_copy(g_hbm.at[srcv], rows, sem).wait()
        pltpu.sync_copy(rows, accsp.at[dstv], add=True)

    plsc.subcore_barrier()
    for j in range(RPT // ZR):
        sl = pl.ds(s * RPT + j * ZR, ZR)
        pltpu.sync_copy(accsp.at[sl], zb)
        pltpu.sync_copy(zb, acc_hbm.at[c, sl])


_sc_edge = pl.kernel(
    _sc_edge_body,
    out_type=jax.ShapeDtypeStruct((NC, NP, D), jnp.float32),
    mesh=_MESH,
    scratch_types=[
        pltpu.VMEM((CH,), jnp.int32),
        pltpu.VMEM((CH,), jnp.int32),
        pltpu.VMEM((CH, D), jnp.float32),
        pltpu.VMEM((ZR, D), jnp.float32),
        pltpu.VMEM_SHARED((NP, D), jnp.float32),
        pltpu.SemaphoreType.DMA,
    ],
)


def _tc_dinv_body(degp_ref, dinv_ref):
    d = degp_ref[0] + degp_ref[1] + 1.0
    dinv_ref[...] = lax.rsqrt(d)


def _tc_dinv(degp2):
    return pl.pallas_call(
        _tc_dinv_body,
        out_shape=jax.ShapeDtypeStruct((N,), jnp.float32),
    )(degp2)


def _tc_pre_body(x_ref, w_ref, dinv_ref, g_ref):
    g = jnp.dot(x_ref[...], w_ref[...], preferred_element_type=jnp.float32)
    g_ref[...] = g * dinv_ref[...]


def _tc_pre(x, w, dinv_c):
    return pl.pallas_call(
        _tc_pre_body,
        out_shape=jax.ShapeDtypeStruct((N, D), jnp.float32),
    )(x, w, dinv_c)


def _tc_mid_body(acc_ref, g_ref, dinv_ref, cvec_ref, w_ref, gout_ref):
    h = (acc_ref[0, :N, :] + acc_ref[1, :N, :] + g_ref[...]) * dinv_ref[...] \
        + cvec_ref[...]
    h = jnp.maximum(h, 0.0)
    gout_ref[...] = jnp.dot(
        h, w_ref[...], preferred_element_type=jnp.float32) * dinv_ref[...]


def _tc_mid(acc, g, dinv_c, cvec, w):
    return pl.pallas_call(
        _tc_mid_body,
        out_shape=jax.ShapeDtypeStruct((N, D), jnp.float32),
    )(acc, g, dinv_c, cvec, w)


def _tc_final_body(acc_ref, g_ref, dinv_ref, cvec_ref, batch_ref, cw0_ref,
                   cb0_ref, cw1_ref, cb1_ref, out_ref, pooled_acc, cnt_acc):
    i = pl.program_id(0)
    h = (acc_ref[0] + acc_ref[1] + g_ref[...]) * dinv_ref[...] + cvec_ref[...]
    bt = batch_ref[0]
    iot = lax.broadcasted_iota(jnp.int32, (B, BR), 0)
    maskT = (iot == bt).astype(jnp.float32)
    pp = jnp.dot(maskT, h, preferred_element_type=jnp.float32)
    cp = jnp.sum(maskT, axis=1, keepdims=True)

    @pl.when(i == 0)
    def _():
        pooled_acc[...] = pp
        cnt_acc[...] = cp

    @pl.when(i != 0)
    def _():
        pooled_acc[...] += pp
        cnt_acc[...] += cp

    @pl.when(i == NBLK - 1)
    def _():
        pm = pooled_acc[...] / jnp.maximum(cnt_acc[...], 1.0)
        z = jnp.dot(pm, cw0_ref[...], preferred_element_type=jnp.float32)
        z = jnp.maximum(z + cb0_ref[...], 0.0)
        sc = jnp.dot(z, cw1_ref[...], preferred_element_type=jnp.float32)
        out_ref[...] = sc + cb1_ref[...]


def _tc_final(acc, g, dinv_c, cvec, batch2, cw0, cb0, cw1b, cb1b):
    return pl.pallas_call(
        _tc_final_body,
        grid=(NBLK,),
        in_specs=[
            pl.BlockSpec((NC, BR, D), lambda i: (0, i, 0)),
            pl.BlockSpec((BR, D), lambda i: (i, 0)),
            pl.BlockSpec((BR, 1), lambda i: (i, 0)),
            pl.BlockSpec((1, D), lambda i: (0, 0)),
            pl.BlockSpec((1, 1, BR), lambda i: (i, 0, 0)),
            pl.BlockSpec((D, H2), lambda i: (0, 0)),
            pl.BlockSpec((1, H2), lambda i: (0, 0)),
            pl.BlockSpec((H2, D), lambda i: (0, 0)),
            pl.BlockSpec((1, D), lambda i: (0, 0)),
        ],
        out_specs=pl.BlockSpec((B, D), lambda i: (0, 0)),
        out_shape=jax.ShapeDtypeStruct((B, D), jnp.float32),
        scratch_shapes=[
            pltpu.VMEM((B, D), jnp.float32),
            pltpu.VMEM((B, 1), jnp.float32),
        ],
    )(acc, g, dinv_c, cvec, batch2, cw0, cb0, cw1b, cb1b)


def kernel(x, edge_index, batch, W0, b0, g0, be0, m0, v0, W1, b1, g1, be1,
           m1, v1, W2, b2, g2, be2, m2, v2, cW0, cb0, cW1, cb1):
    src = edge_index[0].astype(jnp.int32)
    dst = edge_index[1].astype(jnp.int32)

    def fold(W, b, g, be, m, v):
        s_ = g * lax.rsqrt(v + EPS)
        return W * s_[None, :], (b * s_ + be - m * s_).reshape(1, D)

    W0f, c0 = fold(W0, b0, g0, be0, m0, v0)
    W1f, c1 = fold(W1, b1, g1, be1, m1, v1)
    W2f, c2 = fold(W2, b2, g2, be2, m2, v2)

    degp = _sc_deg(dst)
    dinv = _tc_dinv(degp[:, :N, 0])
    dinv_c = dinv.reshape(N, 1)

    g1_ = _tc_pre(x, W0f, dinv_c)
    acc1 = _sc_edge(g1_, src, dst)
    g2_ = _tc_mid(acc1, g1_, dinv_c, c0, W1f)
    acc2 = _sc_edge(g2_, src, dst)
    g3_ = _tc_mid(acc2, g2_, dinv_c, c1, W2f)
    acc3 = _sc_edge(g3_, src, dst)

    out = _tc_final(acc3, g3_, dinv_c, c2,
                    batch.astype(jnp.int32).reshape(NBLK, 1, BR),
                    cW0, cb0.reshape(1, H2),
                    jnp.broadcast_to(cW1, (H2, D)),
                    jnp.broadcast_to(cb1.reshape(1, 1), (1, D)))
    return out[:, 0]


# speedup vs baseline: 14.6436x; 1.3476x over previous
import jax
import jax.numpy as jnp
from jax import lax
from jax.experimental import pallas as pl
from jax.experimental.pallas import tpu as pltpu
from jax.experimental.pallas import tpu_sc as plsc

N = 10000
E = 320000
D = 128
H2 = 64
B = 512
EPS = 1e-5

NC = 2
NS = 16
NW = NC * NS

NP = 10240
EPT = E // NW
CH = 80
NCH = EPT // CH
RPT = NP // NS
ZR = 128

BR = 2000
NBLK = N // BR

_MESH = plsc.VectorSubcoreMesh(core_axis_name="c", subcore_axis_name="s",
                               num_cores=NC, num_subcores=NS)


def _sc_deg_body(dst_hbm, degp_hbm, dstv, onesb, zb, degsp):
    c = lax.axis_index("c")
    s = lax.axis_index("s")
    wid = c * NS + s
    one16 = jnp.ones((16,), jnp.float32)
    zero16 = jnp.zeros((16,), jnp.float32)

    @pl.loop(0, CH)
    def _fill(i):
        for j in range(D // 16):
            onesb[i, pl.ds(j * 16, 16)] = one16

    @pl.loop(0, ZR)
    def _zfill(i):
        for j in range(D // 16):
            zb[i, pl.ds(j * 16, 16)] = zero16

    for j in range(RPT // ZR):
        pltpu.sync_copy(zb, degsp.at[pl.ds(s * RPT + j * ZR, ZR)])
    plsc.subcore_barrier()

    base = wid * EPT

    @pl.loop(0, NCH)
    def _chunk(i):
        pltpu.sync_copy(dst_hbm.at[pl.ds(base + i * CH, CH)], dstv)
        pltpu.sync_copy(onesb, degsp.at[dstv], add=True)

    plsc.subcore_barrier()
    for j in range(RPT // ZR):
        sl = pl.ds(s * RPT + j * ZR, ZR)
        pltpu.sync_copy(degsp.at[sl], zb)
        pltpu.sync_copy(zb, degp_hbm.at[c, sl])


_sc_deg = pl.kernel(
    _sc_deg_body,
    out_type=jax.ShapeDtypeStruct((NC, NP, D), jnp.float32),
    mesh=_MESH,
    scratch_types=[
        pltpu.VMEM((CH,), jnp.int32),
        pltpu.VMEM((CH, D), jnp.float32),
        pltpu.VMEM((ZR, D), jnp.float32),
        pltpu.VMEM_SHARED((NP, D), jnp.float32),
    ],
)


def _sc_edge_body(g_hbm, src_hbm, dst_hbm, acc_hbm, srcall, dstall, srcv_a,
                  dstv_a, rows_a, accsp, sem_a):
    c = lax.axis_index("c")
    s = lax.axis_index("s")
    wid = c * NS + s
    zero16 = jnp.zeros((16,), jnp.float32)

    @pl.loop(0, CH)
    def _zfill(i):
        for j in range(D // 16):
            rows_a[i, pl.ds(j * 16, 16)] = zero16

    for j in range(RPT // CH):
        pltpu.sync_copy(rows_a, accsp.at[pl.ds(s * RPT + j * CH, CH)])

    pltpu.sync_copy(src_hbm.at[pl.ds(wid * EPT, EPT)], srcall)
    pltpu.sync_copy(dst_hbm.at[pl.ds(wid * EPT, EPT)], dstall)
    plsc.subcore_barrier()

    def stage_idx(all_ref, ci, buf):
        for k in range(0, CH - 15, 16):
            buf[pl.ds(k, 16)] = all_ref[pl.ds(ci * CH + k, 16)]
        if CH % 16:
            buf[pl.ds(CH - 16, 16)] = all_ref[pl.ds(ci * CH + CH - 16, 16)]

    @pl.loop(0, NCH)
    def _chunk(i):
        stage_idx(srcall, i, srcv_a)
        pltpu.async_copy(g_hbm.at[srcv_a], rows_a, sem_a).wait()
        stage_idx(dstall, i, dstv_a)
        pltpu.sync_copy(rows_a, accsp.at[dstv_a], add=True)

    plsc.subcore_barrier()
    for j in range(RPT // CH):
        sl = pl.ds(s * RPT + j * CH, CH)
        pltpu.sync_copy(accsp.at[sl], rows_a)
        pltpu.sync_copy(rows_a, acc_hbm.at[c, sl])


_sc_edge = pl.kernel(
    _sc_edge_body,
    out_type=jax.ShapeDtypeStruct((NC, NP, D), jnp.float32),
    mesh=_MESH,
    scratch_types=[
        pltpu.VMEM((EPT,), jnp.int32),
        pltpu.VMEM((EPT,), jnp.int32),
        pltpu.VMEM((CH,), jnp.int32),
        pltpu.VMEM((CH,), jnp.int32),
        pltpu.VMEM((CH, D), jnp.float32),
        pltpu.VMEM_SHARED((NP, D), jnp.float32),
        pltpu.SemaphoreType.DMA,
    ],
)


def _tc_dinv_body(degp_ref, dinv_ref):
    d = degp_ref[0] + degp_ref[1] + 1.0
    dinv_ref[...] = lax.rsqrt(d)


def _tc_dinv(degp2):
    return pl.pallas_call(
        _tc_dinv_body,
        out_shape=jax.ShapeDtypeStruct((N,), jnp.float32),
    )(degp2)


def _tc_pre_body(x_ref, w_ref, dinv_ref, g_ref):
    g = jnp.dot(x_ref[...], w_ref[...], preferred_element_type=jnp.float32)
    g_ref[...] = g * dinv_ref[...]


def _tc_pre(x, w, dinv_c):
    return pl.pallas_call(
        _tc_pre_body,
        out_shape=jax.ShapeDtypeStruct((N, D), jnp.float32),
    )(x, w, dinv_c)


def _tc_mid_body(acc_ref, g_ref, dinv_ref, cvec_ref, w_ref, gout_ref):
    h = (acc_ref[0, :N, :] + acc_ref[1, :N, :] + g_ref[...]) * dinv_ref[...] \
        + cvec_ref[...]
    h = jnp.maximum(h, 0.0)
    gout_ref[...] = jnp.dot(
        h, w_ref[...], preferred_element_type=jnp.float32) * dinv_ref[...]


def _tc_mid(acc, g, dinv_c, cvec, w):
    return pl.pallas_call(
        _tc_mid_body,
        out_shape=jax.ShapeDtypeStruct((N, D), jnp.float32),
    )(acc, g, dinv_c, cvec, w)


def _tc_final_body(acc_ref, g_ref, dinv_ref, cvec_ref, batch_ref, cw0_ref,
                   cb0_ref, cw1_ref, cb1_ref, out_ref, pooled_acc, cnt_acc):
    i = pl.program_id(0)
    h = (acc_ref[0] + acc_ref[1] + g_ref[...]) * dinv_ref[...] + cvec_ref[...]
    bt = batch_ref[0]
    iot = lax.broadcasted_iota(jnp.int32, (B, BR), 0)
    maskT = (iot == bt).astype(jnp.float32)
    pp = jnp.dot(maskT, h, preferred_element_type=jnp.float32)
    cp = jnp.sum(maskT, axis=1, keepdims=True)

    @pl.when(i == 0)
    def _():
        pooled_acc[...] = pp
        cnt_acc[...] = cp

    @pl.when(i != 0)
    def _():
        pooled_acc[...] += pp
        cnt_acc[...] += cp

    @pl.when(i == NBLK - 1)
    def _():
        pm = pooled_acc[...] / jnp.maximum(cnt_acc[...], 1.0)
        z = jnp.dot(pm, cw0_ref[...], preferred_element_type=jnp.float32)
        z = jnp.maximum(z + cb0_ref[...], 0.0)
        sc = jnp.dot(z, cw1_ref[...], preferred_element_type=jnp.float32)
        out_ref[...] = sc + cb1_ref[...]


def _tc_final(acc, g, dinv_c, cvec, batch2, cw0, cb0, cw1b, cb1b):
    return pl.pallas_call(
        _tc_final_body,
        grid=(NBLK,),
        in_specs=[
            pl.BlockSpec((NC, BR, D), lambda i: (0, i, 0)),
            pl.BlockSpec((BR, D), lambda i: (i, 0)),
            pl.BlockSpec((BR, 1), lambda i: (i, 0)),
            pl.BlockSpec((1, D), lambda i: (0, 0)),
            pl.BlockSpec((1, 1, BR), lambda i: (i, 0, 0)),
            pl.BlockSpec((D, H2), lambda i: (0, 0)),
            pl.BlockSpec((1, H2), lambda i: (0, 0)),
            pl.BlockSpec((H2, D), lambda i: (0, 0)),
            pl.BlockSpec((1, D), lambda i: (0, 0)),
        ],
        out_specs=pl.BlockSpec((B, D), lambda i: (0, 0)),
        out_shape=jax.ShapeDtypeStruct((B, D), jnp.float32),
        scratch_shapes=[
            pltpu.VMEM((B, D), jnp.float32),
            pltpu.VMEM((B, 1), jnp.float32),
        ],
    )(acc, g, dinv_c, cvec, batch2, cw0, cb0, cw1b, cb1b)


def kernel(x, edge_index, batch, W0, b0, g0, be0, m0, v0, W1, b1, g1, be1,
           m1, v1, W2, b2, g2, be2, m2, v2, cW0, cb0, cW1, cb1):
    src = edge_index[0].astype(jnp.int32)
    dst = edge_index[1].astype(jnp.int32)

    def fold(W, b, g, be, m, v):
        s_ = g * lax.rsqrt(v + EPS)
        return W * s_[None, :], (b * s_ + be - m * s_).reshape(1, D)

    W0f, c0 = fold(W0, b0, g0, be0, m0, v0)
    W1f, c1 = fold(W1, b1, g1, be1, m1, v1)
    W2f, c2 = fold(W2, b2, g2, be2, m2, v2)

    degp = _sc_deg(dst)
    dinv = _tc_dinv(degp[:, :N, 0])
    dinv_c = dinv.reshape(N, 1)

    g1_ = _tc_pre(x, W0f, dinv_c)
    acc1 = _sc_edge(g1_, src, dst)
    g2_ = _tc_mid(acc1, g1_, dinv_c, c0, W1f)
    acc2 = _sc_edge(g2_, src, dst)
    g3_ = _tc_mid(acc2, g2_, dinv_c, c1, W2f)
    acc3 = _sc_edge(g3_, src, dst)

    out = _tc_final(acc3, g3_, dinv_c, c2,
                    batch.astype(jnp.int32).reshape(NBLK, 1, BR),
                    cW0, cb0.reshape(1, H2),
                    jnp.broadcast_to(cW1, (H2, D)),
                    jnp.broadcast_to(cb1.reshape(1, 1), (1, D)))
    return out[:, 0]
